# Initial kernel scaffold; baseline (speedup 1.0000x reference)
import jax
import jax.numpy as jnp
from jax.experimental import pallas as pl


def kernel(x, w_t, bias):
    raise NotImplementedError("write your pallas kernel here")


# speedup vs baseline: 2.2388x; 2.2388x over previous
import jax
import jax.numpy as jnp
from jax.experimental import pallas as pl
from jax.experimental.pallas import tpu as pltpu


def _mm_kernel(x_ref, w_ref, b_ref, o_ref):
    x_bf = x_ref[...].astype(jnp.bfloat16)
    o_ref[...] = (
        jnp.dot(x_bf, w_ref[...], preferred_element_type=jnp.float32)
        + b_ref[...]
    )


def kernel(x, w_t, bias):
    B, D = x.shape
    D2, C = w_t.shape
    assert D2 == D

    w_bf = w_t.astype(jnp.bfloat16)
    b2 = bias.astype(jnp.float32).reshape(1, C)

    TM = 256
    grid = (B // TM,)

    return pl.pallas_call(
        _mm_kernel,
        out_shape=jax.ShapeDtypeStruct((B, C), jnp.float32),
        grid=grid,
        in_specs=[
            pl.BlockSpec((TM, D), lambda i: (i, 0)),
            pl.BlockSpec((D, C), lambda i: (0, 0)),
            pl.BlockSpec((1, C), lambda i: (0, 0)),
        ],
        out_specs=pl.BlockSpec((TM, C), lambda i: (i, 0)),
        compiler_params=pltpu.CompilerParams(
            dimension_semantics=("parallel",),
            vmem_limit_bytes=64 * 1024 * 1024,
        ),
    )(x, w_bf, b2)


# speedup vs baseline: 2.5067x; 1.1196x over previous
import jax
import jax.numpy as jnp
from jax.experimental import pallas as pl
from jax.experimental.pallas import tpu as pltpu

_TM = 256
_TKC = 256


def _mm_kernel(x_ref, w_ref, b_ref, o_ref, wbf_ref):
    s = pl.program_id(0)
    n_cast = pl.num_programs(0) // 2

    @pl.when(s < n_cast)
    def _cast():
        wbf_ref[pl.ds(s * _TKC, _TKC), :] = w_ref[...].astype(jnp.bfloat16)

    @pl.when(s >= n_cast)
    def _compute():
        x_bf = x_ref[...].astype(jnp.bfloat16)
        o_ref[...] = (
            jnp.dot(x_bf, wbf_ref[...], preferred_element_type=jnp.float32)
            + b_ref[...]
        )


def kernel(x, w_t, bias):
    B, D = x.shape
    D2, C = w_t.shape
    assert D2 == D

    b2 = bias.astype(jnp.float32).reshape(1, C)

    n_cast = D // _TKC
    n_comp = B // _TM
    assert n_cast == n_comp
    grid = (n_cast + n_comp,)

    return pl.pallas_call(
        _mm_kernel,
        out_shape=jax.ShapeDtypeStruct((B, C), jnp.float32),
        grid=grid,
        in_specs=[
            pl.BlockSpec((_TM, D), lambda s: (jnp.maximum(s - n_cast, 0), 0)),
            pl.BlockSpec((_TKC, C), lambda s: (jnp.minimum(s, n_cast - 1), 0)),
            pl.BlockSpec((1, C), lambda s: (0, 0)),
        ],
        out_specs=pl.BlockSpec((_TM, C), lambda s: (jnp.maximum(s - n_cast, 0), 0)),
        scratch_shapes=[pltpu.VMEM((D, C), jnp.bfloat16)],
        compiler_params=pltpu.CompilerParams(
            dimension_semantics=("arbitrary",),
            vmem_limit_bytes=64 * 1024 * 1024,
        ),
    )(x, w_t, b2)


# speedup vs baseline: 2.5336x; 1.0107x over previous
import jax
import jax.numpy as jnp
from jax.experimental import pallas as pl
from jax.experimental.pallas import tpu as pltpu

_TM = 256
_TKW = 512
_NH = 2


def _make_kernel(n_cast, n_comp, cn):
    def _body(x_ref, w_ref, b_ref, o_ref, wbf_ref):
        s = pl.program_id(0)

        @pl.when(s < n_cast)
        def _cast_lo():
            wbf_ref[0, pl.ds(s * _TKW, _TKW), :] = w_ref[...].astype(
                jnp.bfloat16
            )

        @pl.when(jnp.logical_and(s >= n_cast, s < n_cast + n_cast))
        def _cast_hi():
            wbf_ref[1, pl.ds((s - n_cast) * _TKW, _TKW), :] = w_ref[
                ...
            ].astype(jnp.bfloat16)

        @pl.when(jnp.logical_and(s >= n_cast, s < n_cast + n_comp))
        def _pass_a():
            x_bf = x_ref[...].astype(jnp.bfloat16)
            o_ref[...] = (
                jnp.dot(x_bf, wbf_ref[0], preferred_element_type=jnp.float32)
                + b_ref[:, :cn]
            )

        @pl.when(s >= n_cast + n_comp)
        def _pass_b():
            x_bf = x_ref[...].astype(jnp.bfloat16)
            o_ref[...] = (
                jnp.dot(x_bf, wbf_ref[1], preferred_element_type=jnp.float32)
                + b_ref[:, cn:]
            )

    return _body


def kernel(x, w_t, bias):
    B, D = x.shape
    D2, C = w_t.shape
    assert D2 == D
    cn = C // _NH
    n_cast = D // _TKW
    n_comp = B // _TM
    grid = (n_cast + _NH * n_comp,)

    b2 = bias.astype(jnp.float32).reshape(1, C)

    def x_idx(s):
        m = jnp.maximum(s - n_cast, 0)
        return (jnp.where(m >= n_comp, m - n_comp, m), 0)

    def w_idx(s):
        r = jnp.minimum(s, 2 * n_cast - 1)
        return (jnp.where(r >= n_cast, r - n_cast, r),
                jnp.where(r >= n_cast, 1, 0))

    def o_idx(s):
        m = jnp.maximum(s - n_cast, 0)
        return (jnp.where(m >= n_comp, m - n_comp, m),
                jnp.where(m >= n_comp, 1, 0))

    return pl.pallas_call(
        _make_kernel(n_cast, n_comp, cn),
        out_shape=jax.ShapeDtypeStruct((B, C), jnp.float32),
        grid=grid,
        in_specs=[
            pl.BlockSpec((_TM, D), x_idx),
            pl.BlockSpec((_TKW, cn), w_idx),
            pl.BlockSpec((1, C), lambda s: (0, 0)),
        ],
        out_specs=pl.BlockSpec((_TM, cn), o_idx),
        scratch_shapes=[pltpu.VMEM((_NH, D, cn), jnp.bfloat16)],
        compiler_params=pltpu.CompilerParams(
            dimension_semantics=("arbitrary",),
            vmem_limit_bytes=64 * 1024 * 1024,
        ),
    )(x, w_t, b2)


# speedup vs baseline: 2.5635x; 1.0118x over previous
import jax
import jax.numpy as jnp
from jax.experimental import pallas as pl
from jax.experimental.pallas import tpu as pltpu

_TM = 512
_TKW = 512
_NQ = 4


def _make_kernel(n_cast, n_comp, cq):
    n_q = _NQ

    def _body(x_ref, w_ref, b_ref, o_ref, wbf_ref):
        s = pl.program_id(0)

        @pl.when(s < n_cast)
        def _cast_first():
            wbf_ref[0, pl.ds(s * _TKW, _TKW), :] = w_ref[...].astype(
                jnp.bfloat16
            )

        @pl.when(s >= n_cast)
        def _compute_and_cast():
            t = s - n_cast
            q = t // n_comp
            m = t - q * n_comp

            @pl.when(q < n_q - 1)
            def _cast_next():
                r = t - q * n_comp
                wbf_ref[(q + 1) % 2, pl.ds(r * _TKW, _TKW), :] = w_ref[
                    ...
                ].astype(jnp.bfloat16)

            x_bf = x_ref[...].astype(jnp.bfloat16)
            o_ref[...] = (
                jnp.dot(
                    x_bf, wbf_ref[q % 2], preferred_element_type=jnp.float32
                )
                + b_ref[...]
            )

    return _body


def kernel(x, w_t, bias):
    B, D = x.shape
    D2, C = w_t.shape
    assert D2 == D
    cq = C // _NQ
    n_cast = D // _TKW
    n_comp = B // _TM
    assert n_cast == n_comp
    grid = (n_cast + _NQ * n_comp,)

    b2 = bias.astype(jnp.float32).reshape(1, C)

    def x_idx(s):
        t = jnp.maximum(s - n_cast, 0)
        return (t % n_comp, 0)

    def w_idx(s):
        t = jnp.maximum(s - n_cast, 0)
        q_next = jnp.minimum(t // n_comp + 1, _NQ - 1)
        quarter = jnp.where(s < n_cast, 0, q_next)
        r = jnp.where(s < n_cast, s, t % n_comp)
        r = jnp.where(t // n_comp >= _NQ - 1, n_cast - 1, r)
        return (r, quarter)

    def o_idx(s):
        t = jnp.maximum(s - n_cast, 0)
        return (t % n_comp, t // n_comp)

    def b_idx(s):
        t = jnp.maximum(s - n_cast, 0)
        return (0, t // n_comp)

    return pl.pallas_call(
        _make_kernel(n_cast, n_comp, cq),
        out_shape=jax.ShapeDtypeStruct((B, C), jnp.float32),
        grid=grid,
        in_specs=[
            pl.BlockSpec((_TM, D), x_idx),
            pl.BlockSpec((_TKW, cq), w_idx),
            pl.BlockSpec((1, cq), b_idx),
        ],
        out_specs=pl.BlockSpec((_TM, cq), o_idx),
        scratch_shapes=[pltpu.VMEM((2, D, cq), jnp.bfloat16)],
        compiler_params=pltpu.CompilerParams(
            dimension_semantics=("arbitrary",),
            vmem_limit_bytes=64 * 1024 * 1024,
        ),
    )(x, w_t, b2)
